# Initial kernel scaffold; baseline (speedup 1.0000x reference)
import jax
import jax.numpy as jnp
from jax.experimental import pallas as pl


def kernel(input, adj_list, w_bases, w_rel, weight_prev):
    raise NotImplementedError("write your pallas kernel here")


# speedup vs baseline: 1.2468x; 1.2468x over previous
import jax
import jax.numpy as jnp
from jax.experimental import pallas as pl
from jax.experimental.pallas import tpu as pltpu

_N = 4096
_R = 4
_NB = 8
_D_IN = 256
_D_OUT = 256
_BM = 512


def _rgcn_block_kernel(x_ref, wb_ref, wr_ref, adj_ref, o_ref, y_ref):
    m = pl.program_id(0)
    r = pl.program_id(1)

    @pl.when(jnp.logical_and(m == 0, r == 0))
    def _prologue():
        x = x_ref[...]
        wb = wb_ref[...]
        wr = wr_ref[...]
        for rr in range(_R):
            w = jnp.sum(wr[rr].reshape(_NB, 1, 1) * wb, axis=0)
            y_ref[rr] = jnp.dot(x, w, preferred_element_type=jnp.float32)

    acc = jnp.dot(adj_ref[0], y_ref[r], preferred_element_type=jnp.float32)

    @pl.when(r == 0)
    def _init():
        o_ref[...] = acc

    @pl.when(r != 0)
    def _accum():
        o_ref[...] = o_ref[...] + acc


def kernel(input, adj_list, w_bases, w_rel, weight_prev):
    del weight_prev
    grid = (_N // _BM, _R)
    return pl.pallas_call(
        _rgcn_block_kernel,
        grid=grid,
        in_specs=[
            pl.BlockSpec((_N, _D_IN), lambda m, r: (0, 0)),
            pl.BlockSpec((_NB, _D_IN, _D_OUT), lambda m, r: (0, 0, 0)),
            pl.BlockSpec((_R, _NB), lambda m, r: (0, 0)),
            pl.BlockSpec((1, _BM, _N), lambda m, r: (r, m, 0)),
        ],
        out_specs=pl.BlockSpec((_BM, _D_OUT), lambda m, r: (m, 0)),
        out_shape=jax.ShapeDtypeStruct((_N, _D_OUT), jnp.float32),
        scratch_shapes=[pltpu.VMEM((_R, _N, _D_OUT), jnp.float32)],
        compiler_params=pltpu.CompilerParams(
            dimension_semantics=("arbitrary", "arbitrary"),
        ),
    )(input, w_bases, w_rel, adj_list)
